# Initial kernel scaffold; baseline (speedup 1.0000x reference)
import jax
import jax.numpy as jnp
from jax.experimental import pallas as pl


def kernel(atoms, chirals, coordinates, bonds, atype_emb, chiral_emb, apair_w, apair_b, means, stds, bond_emb, lin_W, lin_b):
    raise NotImplementedError("write your pallas kernel here")


# speedup vs baseline: 36.9303x; 36.9303x over previous
import math

import jax
import jax.numpy as jnp
from jax import lax
from jax.experimental import pallas as pl
from jax.experimental.pallas import tpu as pltpu

_B, _L, _D = 32, 256, 512
_AV, _BV, _CV = 128, 32, 4
_PE, _NH = 16, 16
_INV_SQRT_2PI = 1.0 / math.sqrt(2.0 * math.pi)


def _unimol_kernel(ac_ref, ar_ref, cc_ref, xc_ref, xr_ref, yc_ref, yr_ref,
                   zc_ref, zr_ref, bonds_ref, wv_ref, bvv_ref, atype_ref,
                   chiral_ref, means_ref, stds_ref, linw_ref, linb_ref,
                   bemb_ref, emb_ref, pair_ref, g_ref, ohb_ref):
    f32 = jnp.float32
    ac = ac_ref[0]
    ar = ar_ref[0]
    cc = cc_ref[0]
    iota_v = lax.broadcasted_iota(jnp.int32, (_L, _AV), 1)
    a_oh = (ac == iota_v).astype(f32)
    c_oh = (cc == iota_v).astype(f32)

    emb_ref[0] = (jnp.dot(a_oh, atype_ref[...], preferred_element_type=f32)
                  + jnp.dot(c_oh, chiral_ref[...], preferred_element_type=f32))

    aw = jnp.dot(a_oh, wv_ref[...], preferred_element_type=f32)
    ab = jnp.dot(a_oh, bvv_ref[...], preferred_element_type=f32)

    dx = xc_ref[0] - xr_ref[0]
    dy = yc_ref[0] - yr_ref[0]
    dz = zc_ref[0] - zr_ref[0]
    dist = jnp.sqrt(dx * dx + dy * dy + dz * dz + 1e-12)

    nt = (((1,), (1,)), ((), ()))
    for c in range(_PE):
        swc = lax.dot_general(a_oh, aw[:, c * _AV:(c + 1) * _AV], nt,
                              preferred_element_type=f32)
        sbc = lax.dot_general(a_oh, ab[:, c * _AV:(c + 1) * _AV], nt,
                              preferred_element_type=f32)
        s = jnp.abs(stds_ref[c]) + 1e-5
        inv_s = 1.0 / s
        t = (swc * dist + sbc - means_ref[c]) * inv_s
        g_ref[c] = jnp.exp(-0.5 * t * t) * (_INV_SQRT_2PI * inv_s)

    bb = bonds_ref[0]
    for v in range(1, _BV):
        ohb_ref[v] = (bb == v).astype(f32)

    neg_inf = jnp.float32(-jnp.inf)
    maskj = ar == 0
    for h in range(_NH):
        acc = g_ref[0] * linw_ref[h, 0] + linb_ref[h]
        for c in range(1, _PE):
            acc = acc + linw_ref[h, c] * g_ref[c]
        for v in range(1, _BV):
            acc = acc + bemb_ref[v, h] * ohb_ref[v]
        pair_ref[0, h] = jnp.where(maskj, neg_inf, acc)


def kernel(atoms, chirals, coordinates, bonds, atype_emb, chiral_emb,
           apair_w, apair_b, means, stds, bond_emb, lin_W, lin_b):
    f32 = jnp.float32
    atoms = atoms.astype(jnp.int32)
    atoms_col = atoms.reshape(_B, _L, 1)
    atoms_row = atoms.reshape(_B, 1, _L)
    chir_col = chirals.astype(jnp.int32).reshape(_B, _L, 1)
    xc = coordinates[:, :, 0].reshape(_B, _L, 1)
    xr = coordinates[:, :, 0].reshape(_B, 1, _L)
    yc = coordinates[:, :, 1].reshape(_B, _L, 1)
    yr = coordinates[:, :, 1].reshape(_B, 1, _L)
    zc = coordinates[:, :, 2].reshape(_B, _L, 1)
    zr = coordinates[:, :, 2].reshape(_B, 1, _L)
    bonds = bonds.astype(jnp.int32)
    wv = apair_w.reshape(_AV, _AV, _PE).transpose(0, 2, 1).reshape(_AV, _PE * _AV)
    bv = apair_b.reshape(_AV, _AV, _PE).transpose(0, 2, 1).reshape(_AV, _PE * _AV)
    chiral_pad = jnp.zeros((_AV, _D), f32).at[:_CV].set(chiral_emb)

    smem = pl.BlockSpec(memory_space=pltpu.SMEM)
    emb_bd, pair_out = pl.pallas_call(
        _unimol_kernel,
        grid=(_B,),
        in_specs=[
            pl.BlockSpec((1, _L, 1), lambda b: (b, 0, 0)),
            pl.BlockSpec((1, 1, _L), lambda b: (b, 0, 0)),
            pl.BlockSpec((1, _L, 1), lambda b: (b, 0, 0)),
            pl.BlockSpec((1, _L, 1), lambda b: (b, 0, 0)),
            pl.BlockSpec((1, 1, _L), lambda b: (b, 0, 0)),
            pl.BlockSpec((1, _L, 1), lambda b: (b, 0, 0)),
            pl.BlockSpec((1, 1, _L), lambda b: (b, 0, 0)),
            pl.BlockSpec((1, _L, 1), lambda b: (b, 0, 0)),
            pl.BlockSpec((1, 1, _L), lambda b: (b, 0, 0)),
            pl.BlockSpec((1, _L, _L), lambda b: (b, 0, 0)),
            pl.BlockSpec((_AV, _PE * _AV), lambda b: (0, 0)),
            pl.BlockSpec((_AV, _PE * _AV), lambda b: (0, 0)),
            pl.BlockSpec((_AV, _D), lambda b: (0, 0)),
            pl.BlockSpec((_AV, _D), lambda b: (0, 0)),
            smem,
            smem,
            smem,
            smem,
            smem,
        ],
        out_specs=[
            pl.BlockSpec((1, _L, _D), lambda b: (b, 0, 0)),
            pl.BlockSpec((1, _NH, _L, _L), lambda b: (b, 0, 0, 0)),
        ],
        out_shape=[
            jax.ShapeDtypeStruct((_B, _L, _D), f32),
            jax.ShapeDtypeStruct((_B, _NH, _L, _L), f32),
        ],
        scratch_shapes=[
            pltpu.VMEM((_PE, _L, _L), f32),
            pltpu.VMEM((_BV, _L, _L), f32),
        ],
        compiler_params=pltpu.CompilerParams(dimension_semantics=("parallel",)),
    )(atoms_col, atoms_row, chir_col, xc, xr, yc, yr, zc, zr, bonds, wv, bv,
      atype_emb, chiral_pad, means, stds, lin_W, lin_b, bond_emb)
    atoms_emb = jnp.transpose(emb_bd, (1, 0, 2))
    return atoms_emb, pair_out


# speedup vs baseline: 37.5530x; 1.0169x over previous
import functools
import math

import jax
import jax.numpy as jnp
from jax import lax
from jax.experimental import pallas as pl
from jax.experimental.pallas import tpu as pltpu
from jax.experimental.pallas import tpu_sc as plsc

_B, _L, _D = 32, 256, 512
_AV, _BV, _CV = 128, 32, 4
_PE, _NH = 16, 16
_INV_SQRT_2PI = 1.0 / math.sqrt(2.0 * math.pi)


def _pair_kernel(ac_ref, ar_ref, xc_ref, xr_ref, yc_ref, yr_ref,
                 zc_ref, zr_ref, bonds_ref, wv_ref, bvv_ref,
                 means_ref, stds_ref, linw_ref, linb_ref,
                 bemb_ref, pair_ref, g_ref, ohb_ref):
    f32 = jnp.float32
    ac = ac_ref[0]
    ar = ar_ref[0]
    iota_v = lax.broadcasted_iota(jnp.int32, (_L, _AV), 1)
    a_oh = (ac == iota_v).astype(f32)

    aw = jnp.dot(a_oh, wv_ref[...], preferred_element_type=f32)
    ab = jnp.dot(a_oh, bvv_ref[...], preferred_element_type=f32)

    dx = xc_ref[0] - xr_ref[0]
    dy = yc_ref[0] - yr_ref[0]
    dz = zc_ref[0] - zr_ref[0]
    dist = jnp.sqrt(dx * dx + dy * dy + dz * dz + 1e-12)

    nt = (((1,), (1,)), ((), ()))
    for c in range(_PE):
        swc = lax.dot_general(a_oh, aw[:, c * _AV:(c + 1) * _AV], nt,
                              preferred_element_type=f32)
        sbc = lax.dot_general(a_oh, ab[:, c * _AV:(c + 1) * _AV], nt,
                              preferred_element_type=f32)
        s = jnp.abs(stds_ref[c]) + 1e-5
        inv_s = 1.0 / s
        t = (swc * dist + sbc - means_ref[c]) * inv_s
        g_ref[c] = jnp.exp(-0.5 * t * t) * (_INV_SQRT_2PI * inv_s)

    bb = bonds_ref[0]
    for v in range(1, _BV):
        ohb_ref[v] = (bb == v).astype(f32)

    neg_inf = jnp.float32(-jnp.inf)
    maskj = ar == 0
    for h in range(_NH):
        acc = g_ref[0] * linw_ref[h, 0] + linb_ref[h]
        for c in range(1, _PE):
            acc = acc + linw_ref[h, c] * g_ref[c]
        for v in range(1, _BV):
            acc = acc + bemb_ref[v, h] * ohb_ref[v]
        pair_ref[0, h] = jnp.where(maskj, neg_inf, acc)


def _emb_sc_kernel(table_ref, a_ref, c_ref, out_ref, a_v, c_v, idx_v,
                   rows_v, sem):
    nc = 2
    wid = lax.axis_index("s") * nc + lax.axis_index("c")
    base = wid * _ROWS_PER_W
    for chunk in range(_ROWS_PER_W // _CHUNK):
        off = base + chunk * _CHUNK
        pltpu.sync_copy(a_ref.at[pl.ds(off, _CHUNK)], a_v)
        pltpu.sync_copy(c_ref.at[pl.ds(off, _CHUNK)], c_v)
        for i in range(_CHUNK // 16):
            s = pl.ds(i * 16, 16)
            idx_v[s] = a_v[s] * _CV + c_v[s]
        pltpu.async_copy(table_ref.at[idx_v], rows_v, sem).wait()
        pltpu.sync_copy(rows_v, out_ref.at[pl.ds(off, _CHUNK)])


_ROWS_PER_W = (_B * _L) // 32
_CHUNK = 128


def _emb_lookup_sc(table, atoms_lb, chirals_lb):
    mesh = plsc.VectorSubcoreMesh(core_axis_name="c", subcore_axis_name="s")
    return pl.kernel(
        _emb_sc_kernel,
        mesh=mesh,
        out_type=jax.ShapeDtypeStruct((_B * _L, _D), jnp.float32),
        scratch_types=[
            pltpu.VMEM((_CHUNK,), jnp.int32),
            pltpu.VMEM((_CHUNK,), jnp.int32),
            pltpu.VMEM((_CHUNK,), jnp.int32),
            pltpu.VMEM((_CHUNK, _D), jnp.float32),
            pltpu.SemaphoreType.DMA,
        ],
    )(table, atoms_lb, chirals_lb)


def kernel(atoms, chirals, coordinates, bonds, atype_emb, chiral_emb,
           apair_w, apair_b, means, stds, bond_emb, lin_W, lin_b):
    f32 = jnp.float32
    atoms = atoms.astype(jnp.int32)
    chirals = chirals.astype(jnp.int32)
    atoms_col = atoms.reshape(_B, _L, 1)
    atoms_row = atoms.reshape(_B, 1, _L)
    xc = coordinates[:, :, 0].reshape(_B, _L, 1)
    xr = coordinates[:, :, 0].reshape(_B, 1, _L)
    yc = coordinates[:, :, 1].reshape(_B, _L, 1)
    yr = coordinates[:, :, 1].reshape(_B, 1, _L)
    zc = coordinates[:, :, 2].reshape(_B, _L, 1)
    zr = coordinates[:, :, 2].reshape(_B, 1, _L)
    bonds = bonds.astype(jnp.int32)
    wv = apair_w.reshape(_AV, _AV, _PE).transpose(0, 2, 1).reshape(_AV, _PE * _AV)
    bv = apair_b.reshape(_AV, _AV, _PE).transpose(0, 2, 1).reshape(_AV, _PE * _AV)

    smem = pl.BlockSpec(memory_space=pltpu.SMEM)
    pair_out = pl.pallas_call(
        _pair_kernel,
        grid=(_B,),
        in_specs=[
            pl.BlockSpec((1, _L, 1), lambda b: (b, 0, 0)),
            pl.BlockSpec((1, 1, _L), lambda b: (b, 0, 0)),
            pl.BlockSpec((1, _L, 1), lambda b: (b, 0, 0)),
            pl.BlockSpec((1, 1, _L), lambda b: (b, 0, 0)),
            pl.BlockSpec((1, _L, 1), lambda b: (b, 0, 0)),
            pl.BlockSpec((1, 1, _L), lambda b: (b, 0, 0)),
            pl.BlockSpec((1, _L, 1), lambda b: (b, 0, 0)),
            pl.BlockSpec((1, 1, _L), lambda b: (b, 0, 0)),
            pl.BlockSpec((1, _L, _L), lambda b: (b, 0, 0)),
            pl.BlockSpec((_AV, _PE * _AV), lambda b: (0, 0)),
            pl.BlockSpec((_AV, _PE * _AV), lambda b: (0, 0)),
            smem,
            smem,
            smem,
            smem,
            smem,
        ],
        out_specs=pl.BlockSpec((1, _NH, _L, _L), lambda b: (b, 0, 0, 0)),
        out_shape=jax.ShapeDtypeStruct((_B, _NH, _L, _L), f32),
        scratch_shapes=[
            pltpu.VMEM((_PE, _L, _L), f32),
            pltpu.VMEM((_BV, _L, _L), f32),
        ],
        compiler_params=pltpu.CompilerParams(dimension_semantics=("parallel",)),
    )(atoms_col, atoms_row, xc, xr, yc, yr, zc, zr, bonds, wv, bv,
      means, stds, lin_W, lin_b, bond_emb)

    table = (atype_emb[:, None, :] + chiral_emb[None, :, :]).reshape(
        _AV * _CV, _D)
    atoms_lb = atoms.T.reshape(_B * _L)
    chirals_lb = chirals.T.reshape(_B * _L)
    emb_rows = _emb_lookup_sc(table, atoms_lb, chirals_lb)
    atoms_emb = emb_rows.reshape(_L, _B, _D)
    return atoms_emb, pair_out


# speedup vs baseline: 78.8552x; 2.0998x over previous
import functools
import math

import jax
import jax.numpy as jnp
from jax import lax
from jax.experimental import pallas as pl
from jax.experimental.pallas import tpu as pltpu
from jax.experimental.pallas import tpu_sc as plsc

_B, _L, _D = 32, 256, 512
_AV, _BV, _CV = 128, 32, 4
_PE, _NH = 16, 16
_INV_SQRT_2PI = 1.0 / math.sqrt(2.0 * math.pi)


def _pair_kernel(ac_ref, ar_ref, xc_ref, xr_ref, yc_ref, yr_ref,
                 zc_ref, zr_ref, bonds_ref, wv_ref, bvv_ref, bembt_ref,
                 means_ref, stds_ref, linw_ref, linb_ref,
                 pair_ref, g_ref):
    f32 = jnp.float32
    ac = ac_ref[0]
    ar = ar_ref[0]
    iota_v = lax.broadcasted_iota(jnp.int32, (_L, _AV), 1)
    a_oh = (ac == iota_v).astype(f32)

    aw = jnp.dot(a_oh, wv_ref[...], preferred_element_type=f32)
    ab = jnp.dot(a_oh, bvv_ref[...], preferred_element_type=f32)

    dx = xc_ref[0] - xr_ref[0]
    dy = yc_ref[0] - yr_ref[0]
    dz = zc_ref[0] - zr_ref[0]
    dist = jnp.sqrt(dx * dx + dy * dy + dz * dz + 1e-12)

    nt = (((1,), (1,)), ((), ()))
    for c in range(_PE):
        swc = lax.dot_general(a_oh, aw[:, c * _AV:(c + 1) * _AV], nt,
                              preferred_element_type=f32)
        sbc = lax.dot_general(a_oh, ab[:, c * _AV:(c + 1) * _AV], nt,
                              preferred_element_type=f32)
        s = jnp.abs(stds_ref[c]) + 1e-5
        inv_s = 1.0 / s
        t = (swc * dist + sbc - means_ref[c]) * inv_s
        g_ref[c] = jnp.exp(-0.5 * t * t) * (_INV_SQRT_2PI * inv_s)

    bb = bonds_ref[0]
    neg_inf = jnp.float32(-jnp.inf)
    maskj = ar == 0
    for h in range(_NH):
        acc = g_ref[0] * linw_ref[h, 0] + linb_ref[h]
        for c in range(1, _PE):
            acc = acc + linw_ref[h, c] * g_ref[c]
        lut = jnp.broadcast_to(bembt_ref[h:h + 1, :], (_L, _BV))
        acc = acc + jnp.take_along_axis(
            lut, bb, axis=1, mode=lax.GatherScatterMode.PROMISE_IN_BOUNDS)
        pair_ref[0, h] = jnp.where(maskj, neg_inf, acc)


def _emb_sc_kernel(table_ref, a_ref, c_ref, out_ref, a_v, c_v, idx_v,
                   rows_v, sem):
    nc = 2
    wid = lax.axis_index("s") * nc + lax.axis_index("c")
    base = wid * _ROWS_PER_W
    for chunk in range(_ROWS_PER_W // _CHUNK):
        off = base + chunk * _CHUNK
        pltpu.sync_copy(a_ref.at[pl.ds(off, _CHUNK)], a_v)
        pltpu.sync_copy(c_ref.at[pl.ds(off, _CHUNK)], c_v)
        for i in range(_CHUNK // 16):
            s = pl.ds(i * 16, 16)
            idx_v[s] = a_v[s] * _CV + c_v[s]
        pltpu.async_copy(table_ref.at[idx_v], rows_v, sem).wait()
        pltpu.sync_copy(rows_v, out_ref.at[pl.ds(off, _CHUNK)])


_ROWS_PER_W = (_B * _L) // 32
_CHUNK = 128


def _emb_lookup_sc(table, atoms_lb, chirals_lb):
    mesh = plsc.VectorSubcoreMesh(core_axis_name="c", subcore_axis_name="s")
    return pl.kernel(
        _emb_sc_kernel,
        mesh=mesh,
        out_type=jax.ShapeDtypeStruct((_B * _L, _D), jnp.float32),
        scratch_types=[
            pltpu.VMEM((_CHUNK,), jnp.int32),
            pltpu.VMEM((_CHUNK,), jnp.int32),
            pltpu.VMEM((_CHUNK,), jnp.int32),
            pltpu.VMEM((_CHUNK, _D), jnp.float32),
            pltpu.SemaphoreType.DMA,
        ],
    )(table, atoms_lb, chirals_lb)


def kernel(atoms, chirals, coordinates, bonds, atype_emb, chiral_emb,
           apair_w, apair_b, means, stds, bond_emb, lin_W, lin_b):
    f32 = jnp.float32
    atoms = atoms.astype(jnp.int32)
    chirals = chirals.astype(jnp.int32)
    atoms_col = atoms.reshape(_B, _L, 1)
    atoms_row = atoms.reshape(_B, 1, _L)
    xc = coordinates[:, :, 0].reshape(_B, _L, 1)
    xr = coordinates[:, :, 0].reshape(_B, 1, _L)
    yc = coordinates[:, :, 1].reshape(_B, _L, 1)
    yr = coordinates[:, :, 1].reshape(_B, 1, _L)
    zc = coordinates[:, :, 2].reshape(_B, _L, 1)
    zr = coordinates[:, :, 2].reshape(_B, 1, _L)
    bonds = bonds.astype(jnp.int32)
    wv = apair_w.reshape(_AV, _AV, _PE).transpose(0, 2, 1).reshape(_AV, _PE * _AV)
    bv = apair_b.reshape(_AV, _AV, _PE).transpose(0, 2, 1).reshape(_AV, _PE * _AV)

    smem = pl.BlockSpec(memory_space=pltpu.SMEM)
    pair_out = pl.pallas_call(
        _pair_kernel,
        grid=(_B,),
        in_specs=[
            pl.BlockSpec((1, _L, 1), lambda b: (b, 0, 0)),
            pl.BlockSpec((1, 1, _L), lambda b: (b, 0, 0)),
            pl.BlockSpec((1, _L, 1), lambda b: (b, 0, 0)),
            pl.BlockSpec((1, 1, _L), lambda b: (b, 0, 0)),
            pl.BlockSpec((1, _L, 1), lambda b: (b, 0, 0)),
            pl.BlockSpec((1, 1, _L), lambda b: (b, 0, 0)),
            pl.BlockSpec((1, _L, 1), lambda b: (b, 0, 0)),
            pl.BlockSpec((1, 1, _L), lambda b: (b, 0, 0)),
            pl.BlockSpec((1, _L, _L), lambda b: (b, 0, 0)),
            pl.BlockSpec((_AV, _PE * _AV), lambda b: (0, 0)),
            pl.BlockSpec((_AV, _PE * _AV), lambda b: (0, 0)),
            pl.BlockSpec((_NH, _BV), lambda b: (0, 0)),
            smem,
            smem,
            smem,
            smem,
        ],
        out_specs=pl.BlockSpec((1, _NH, _L, _L), lambda b: (b, 0, 0, 0)),
        out_shape=jax.ShapeDtypeStruct((_B, _NH, _L, _L), f32),
        scratch_shapes=[
            pltpu.VMEM((_PE, _L, _L), f32),
        ],
        compiler_params=pltpu.CompilerParams(dimension_semantics=("parallel",)),
    )(atoms_col, atoms_row, xc, xr, yc, yr, zc, zr, bonds, wv, bv,
      bond_emb.T, means, stds, lin_W, lin_b)

    table = (atype_emb[:, None, :] + chiral_emb[None, :, :]).reshape(
        _AV * _CV, _D)
    atoms_lb = atoms.T.reshape(_B * _L)
    chirals_lb = chirals.T.reshape(_B * _L)
    emb_rows = _emb_lookup_sc(table, atoms_lb, chirals_lb)
    atoms_emb = emb_rows.reshape(_L, _B, _D)
    return atoms_emb, pair_out


# speedup vs baseline: 119.3728x; 1.5138x over previous
import functools
import math

import jax
import jax.numpy as jnp
from jax import lax
from jax.experimental import pallas as pl
from jax.experimental.pallas import tpu as pltpu
from jax.experimental.pallas import tpu_sc as plsc

_B, _L, _D = 32, 256, 512
_AV, _BV, _CV = 128, 32, 4
_PE, _NH = 16, 16
_INV_SQRT_2PI = 1.0 / math.sqrt(2.0 * math.pi)


def _pair_kernel(ac_ref, ar_ref, xc_ref, xr_ref, yc_ref, yr_ref,
                 zc_ref, zr_ref, bonds_ref, wv_ref, bvv_ref, bembt_ref,
                 linw_vm_ref, means_ref, stds_ref, linb_ref,
                 pair_ref, g_ref):
    f32 = jnp.float32
    ac = ac_ref[0]
    ar = ar_ref[0]
    iota_v = lax.broadcasted_iota(jnp.int32, (_L, _AV), 1)
    a_oh = (ac == iota_v).astype(f32)

    aw = jnp.dot(a_oh, wv_ref[...], preferred_element_type=f32)
    ab = jnp.dot(a_oh, bvv_ref[...], preferred_element_type=f32)

    dx = xc_ref[0] - xr_ref[0]
    dy = yc_ref[0] - yr_ref[0]
    dz = zc_ref[0] - zr_ref[0]
    dist = jnp.sqrt(dx * dx + dy * dy + dz * dz + 1e-12)

    nt = (((1,), (1,)), ((), ()))
    for c in range(_PE):
        swc = lax.dot_general(a_oh, aw[:, c * _AV:(c + 1) * _AV], nt,
                              preferred_element_type=f32)
        sbc = lax.dot_general(a_oh, ab[:, c * _AV:(c + 1) * _AV], nt,
                              preferred_element_type=f32)
        s = jnp.abs(stds_ref[c]) + 1e-5
        inv_s = 1.0 / s
        t = (swc * dist + sbc - means_ref[c]) * inv_s
        g_ref[c] = jnp.exp(-0.5 * t * t) * (_INV_SQRT_2PI * inv_s)

    bb = bonds_ref[0]
    neg_inf = jnp.float32(-jnp.inf)
    maskj = ar == 0
    heads = lax.dot_general(linw_vm_ref[...], g_ref[...],
                            (((1,), (0,)), ((), ())),
                            preferred_element_type=f32)
    for h in range(_NH):
        acc = heads[h] + linb_ref[h]
        lut = jnp.broadcast_to(bembt_ref[h:h + 1, :], (_L, _BV))
        acc = acc + jnp.take_along_axis(
            lut, bb, axis=1, mode=lax.GatherScatterMode.PROMISE_IN_BOUNDS)
        pair_ref[0, h] = jnp.where(maskj, neg_inf, acc)


def _emb_sc_kernel(table_ref, a_ref, c_ref, out_ref, a_v, c_v, idx_v,
                   rows_v, sem):
    nc = 2
    wid = lax.axis_index("s") * nc + lax.axis_index("c")
    base = wid * _ROWS_PER_W
    for chunk in range(_ROWS_PER_W // _CHUNK):
        off = base + chunk * _CHUNK
        pltpu.sync_copy(a_ref.at[pl.ds(off, _CHUNK)], a_v)
        pltpu.sync_copy(c_ref.at[pl.ds(off, _CHUNK)], c_v)
        for i in range(_CHUNK // 16):
            s = pl.ds(i * 16, 16)
            idx_v[s] = a_v[s] * _CV + c_v[s]
        pltpu.async_copy(table_ref.at[idx_v], rows_v, sem).wait()
        pltpu.sync_copy(rows_v, out_ref.at[pl.ds(off, _CHUNK)])


_ROWS_PER_W = (_B * _L) // 32
_CHUNK = 128


def _emb_lookup_sc(table, atoms_lb, chirals_lb):
    mesh = plsc.VectorSubcoreMesh(core_axis_name="c", subcore_axis_name="s")
    return pl.kernel(
        _emb_sc_kernel,
        mesh=mesh,
        out_type=jax.ShapeDtypeStruct((_B * _L, _D), jnp.float32),
        scratch_types=[
            pltpu.VMEM((_CHUNK,), jnp.int32),
            pltpu.VMEM((_CHUNK,), jnp.int32),
            pltpu.VMEM((_CHUNK,), jnp.int32),
            pltpu.VMEM((_CHUNK, _D), jnp.float32),
            pltpu.SemaphoreType.DMA,
        ],
    )(table, atoms_lb, chirals_lb)


def kernel(atoms, chirals, coordinates, bonds, atype_emb, chiral_emb,
           apair_w, apair_b, means, stds, bond_emb, lin_W, lin_b):
    f32 = jnp.float32
    atoms = atoms.astype(jnp.int32)
    chirals = chirals.astype(jnp.int32)
    atoms_col = atoms.reshape(_B, _L, 1)
    atoms_row = atoms.reshape(_B, 1, _L)
    xc = coordinates[:, :, 0].reshape(_B, _L, 1)
    xr = coordinates[:, :, 0].reshape(_B, 1, _L)
    yc = coordinates[:, :, 1].reshape(_B, _L, 1)
    yr = coordinates[:, :, 1].reshape(_B, 1, _L)
    zc = coordinates[:, :, 2].reshape(_B, _L, 1)
    zr = coordinates[:, :, 2].reshape(_B, 1, _L)
    bonds = bonds.astype(jnp.int32)
    wv = apair_w.reshape(_AV, _AV, _PE).transpose(0, 2, 1).reshape(_AV, _PE * _AV)
    bv = apair_b.reshape(_AV, _AV, _PE).transpose(0, 2, 1).reshape(_AV, _PE * _AV)

    smem = pl.BlockSpec(memory_space=pltpu.SMEM)
    pair_out = pl.pallas_call(
        _pair_kernel,
        grid=(_B,),
        in_specs=[
            pl.BlockSpec((1, _L, 1), lambda b: (b, 0, 0)),
            pl.BlockSpec((1, 1, _L), lambda b: (b, 0, 0)),
            pl.BlockSpec((1, _L, 1), lambda b: (b, 0, 0)),
            pl.BlockSpec((1, 1, _L), lambda b: (b, 0, 0)),
            pl.BlockSpec((1, _L, 1), lambda b: (b, 0, 0)),
            pl.BlockSpec((1, 1, _L), lambda b: (b, 0, 0)),
            pl.BlockSpec((1, _L, 1), lambda b: (b, 0, 0)),
            pl.BlockSpec((1, 1, _L), lambda b: (b, 0, 0)),
            pl.BlockSpec((1, _L, _L), lambda b: (b, 0, 0)),
            pl.BlockSpec((_AV, _PE * _AV), lambda b: (0, 0)),
            pl.BlockSpec((_AV, _PE * _AV), lambda b: (0, 0)),
            pl.BlockSpec((_NH, _BV), lambda b: (0, 0)),
            pl.BlockSpec((_NH, _PE), lambda b: (0, 0)),
            smem,
            smem,
            smem,
        ],
        out_specs=pl.BlockSpec((1, _NH, _L, _L), lambda b: (b, 0, 0, 0)),
        out_shape=jax.ShapeDtypeStruct((_B, _NH, _L, _L), f32),
        scratch_shapes=[
            pltpu.VMEM((_PE, _L, _L), f32),
        ],
        compiler_params=pltpu.CompilerParams(dimension_semantics=("parallel",)),
    )(atoms_col, atoms_row, xc, xr, yc, yr, zc, zr, bonds, wv, bv,
      bond_emb.T, lin_W, means, stds, lin_b)

    table = (atype_emb[:, None, :] + chiral_emb[None, :, :]).reshape(
        _AV * _CV, _D)
    atoms_lb = atoms.T.reshape(_B * _L)
    chirals_lb = chirals.T.reshape(_B * _L)
    emb_rows = _emb_lookup_sc(table, atoms_lb, chirals_lb)
    atoms_emb = emb_rows.reshape(_L, _B, _D)
    return atoms_emb, pair_out


# speedup vs baseline: 119.8636x; 1.0041x over previous
import functools
import math

import jax
import jax.numpy as jnp
from jax import lax
from jax.experimental import pallas as pl
from jax.experimental.pallas import tpu as pltpu
from jax.experimental.pallas import tpu_sc as plsc

_B, _L, _D = 32, 256, 512
_AV, _BV, _CV = 128, 32, 4
_PE, _NH = 16, 16
_INV_SQRT_2PI = 1.0 / math.sqrt(2.0 * math.pi)


def _pair_kernel(ac_ref, ar_ref, xc_ref, xr_ref, yc_ref, yr_ref,
                 zc_ref, zr_ref, bonds_ref, wv_ref, bvv_ref, bembt_ref,
                 linw_vm_ref, means_ref, stds_ref, linb_ref,
                 pair_ref, g_ref):
    f32 = jnp.float32
    ac = ac_ref[0]
    ar = ar_ref[0]
    iota_v = lax.broadcasted_iota(jnp.int32, (_L, _AV), 1)
    a_oh = (ac == iota_v).astype(f32)

    aw = jnp.dot(a_oh, wv_ref[...], preferred_element_type=f32)
    ab = jnp.dot(a_oh, bvv_ref[...], preferred_element_type=f32)

    dx = xc_ref[0] - xr_ref[0]
    dy = yc_ref[0] - yr_ref[0]
    dz = zc_ref[0] - zr_ref[0]
    dist = jnp.sqrt(dx * dx + dy * dy + dz * dz + 1e-12)

    nt = (((1,), (1,)), ((), ()))
    a_oh16 = a_oh.astype(jnp.bfloat16)
    for c in range(_PE):
        x = jnp.concatenate([aw[:, c * _AV:(c + 1) * _AV],
                             ab[:, c * _AV:(c + 1) * _AV]], axis=0)
        y = lax.dot_general(a_oh16, x.astype(jnp.bfloat16), nt,
                            preferred_element_type=f32)
        swc = y[:, :_L]
        sbc = y[:, _L:]
        s = jnp.abs(stds_ref[c]) + 1e-5
        inv_s = 1.0 / s
        t = (swc * dist + sbc - means_ref[c]) * inv_s
        g_ref[c] = jnp.exp(-0.5 * t * t) * (_INV_SQRT_2PI * inv_s)

    bb = bonds_ref[0]
    neg_inf = jnp.float32(-jnp.inf)
    maskj = ar == 0
    heads = lax.dot_general(linw_vm_ref[...], g_ref[...],
                            (((1,), (0,)), ((), ())),
                            preferred_element_type=f32)
    for h in range(_NH):
        acc = heads[h] + linb_ref[h]
        lut = jnp.broadcast_to(bembt_ref[h:h + 1, :], (_L, _BV))
        acc = acc + jnp.take_along_axis(
            lut, bb, axis=1, mode=lax.GatherScatterMode.PROMISE_IN_BOUNDS)
        pair_ref[0, h] = jnp.where(maskj, neg_inf, acc)


def _emb_sc_kernel(table_ref, a_ref, c_ref, out_ref, a_v, c_v, idx_v,
                   rows_v, sem):
    nc = 2
    wid = lax.axis_index("s") * nc + lax.axis_index("c")
    base = wid * _ROWS_PER_W
    for chunk in range(_ROWS_PER_W // _CHUNK):
        off = base + chunk * _CHUNK
        pltpu.sync_copy(a_ref.at[pl.ds(off, _CHUNK)], a_v)
        pltpu.sync_copy(c_ref.at[pl.ds(off, _CHUNK)], c_v)
        for i in range(_CHUNK // 16):
            s = pl.ds(i * 16, 16)
            idx_v[s] = a_v[s] * _CV + c_v[s]
        pltpu.async_copy(table_ref.at[idx_v], rows_v, sem).wait()
        pltpu.sync_copy(rows_v, out_ref.at[pl.ds(off, _CHUNK)])


_ROWS_PER_W = (_B * _L) // 32
_CHUNK = 128


def _emb_lookup_sc(table, atoms_lb, chirals_lb):
    mesh = plsc.VectorSubcoreMesh(core_axis_name="c", subcore_axis_name="s")
    return pl.kernel(
        _emb_sc_kernel,
        mesh=mesh,
        out_type=jax.ShapeDtypeStruct((_B * _L, _D), jnp.float32),
        scratch_types=[
            pltpu.VMEM((_CHUNK,), jnp.int32),
            pltpu.VMEM((_CHUNK,), jnp.int32),
            pltpu.VMEM((_CHUNK,), jnp.int32),
            pltpu.VMEM((_CHUNK, _D), jnp.float32),
            pltpu.SemaphoreType.DMA,
        ],
    )(table, atoms_lb, chirals_lb)


def kernel(atoms, chirals, coordinates, bonds, atype_emb, chiral_emb,
           apair_w, apair_b, means, stds, bond_emb, lin_W, lin_b):
    f32 = jnp.float32
    atoms = atoms.astype(jnp.int32)
    chirals = chirals.astype(jnp.int32)
    atoms_col = atoms.reshape(_B, _L, 1)
    atoms_row = atoms.reshape(_B, 1, _L)
    xc = coordinates[:, :, 0].reshape(_B, _L, 1)
    xr = coordinates[:, :, 0].reshape(_B, 1, _L)
    yc = coordinates[:, :, 1].reshape(_B, _L, 1)
    yr = coordinates[:, :, 1].reshape(_B, 1, _L)
    zc = coordinates[:, :, 2].reshape(_B, _L, 1)
    zr = coordinates[:, :, 2].reshape(_B, 1, _L)
    bonds = bonds.astype(jnp.int32)
    wv = apair_w.reshape(_AV, _AV, _PE).transpose(0, 2, 1).reshape(_AV, _PE * _AV)
    bv = apair_b.reshape(_AV, _AV, _PE).transpose(0, 2, 1).reshape(_AV, _PE * _AV)

    smem = pl.BlockSpec(memory_space=pltpu.SMEM)
    pair_out = pl.pallas_call(
        _pair_kernel,
        grid=(_B,),
        in_specs=[
            pl.BlockSpec((1, _L, 1), lambda b: (b, 0, 0)),
            pl.BlockSpec((1, 1, _L), lambda b: (b, 0, 0)),
            pl.BlockSpec((1, _L, 1), lambda b: (b, 0, 0)),
            pl.BlockSpec((1, 1, _L), lambda b: (b, 0, 0)),
            pl.BlockSpec((1, _L, 1), lambda b: (b, 0, 0)),
            pl.BlockSpec((1, 1, _L), lambda b: (b, 0, 0)),
            pl.BlockSpec((1, _L, 1), lambda b: (b, 0, 0)),
            pl.BlockSpec((1, 1, _L), lambda b: (b, 0, 0)),
            pl.BlockSpec((1, _L, _L), lambda b: (b, 0, 0)),
            pl.BlockSpec((_AV, _PE * _AV), lambda b: (0, 0)),
            pl.BlockSpec((_AV, _PE * _AV), lambda b: (0, 0)),
            pl.BlockSpec((_NH, _BV), lambda b: (0, 0)),
            pl.BlockSpec((_NH, _PE), lambda b: (0, 0)),
            smem,
            smem,
            smem,
        ],
        out_specs=pl.BlockSpec((1, _NH, _L, _L), lambda b: (b, 0, 0, 0)),
        out_shape=jax.ShapeDtypeStruct((_B, _NH, _L, _L), f32),
        scratch_shapes=[
            pltpu.VMEM((_PE, _L, _L), f32),
        ],
        compiler_params=pltpu.CompilerParams(dimension_semantics=("parallel",)),
    )(atoms_col, atoms_row, xc, xr, yc, yr, zc, zr, bonds, wv, bv,
      bond_emb.T, lin_W, means, stds, lin_b)

    table = (atype_emb[:, None, :] + chiral_emb[None, :, :]).reshape(
        _AV * _CV, _D)
    atoms_lb = atoms.T.reshape(_B * _L)
    chirals_lb = chirals.T.reshape(_B * _L)
    emb_rows = _emb_lookup_sc(table, atoms_lb, chirals_lb)
    atoms_emb = emb_rows.reshape(_L, _B, _D)
    return atoms_emb, pair_out
